# baseline (speedup 1.0000x reference)
import functools

import jax
import jax.numpy as jnp
from jax import lax
from jax.experimental import pallas as pl
from jax.experimental.pallas import tpu as pltpu
from jax.experimental.pallas import tpu_sc as plsc

B = 4096
N_DISC, N_CONT = 6, 4
N_FEAT = N_DISC + N_CONT
V1 = 100001
D = 32
NC, NS = 2, 16
NW = NC * NS
BW = B // NW
NK = BW // 16


def _body(qoe_idx, ch_idx, fu_idx, qoe_cont, ch_cont, fu_cont,
          qoe_tab, ch_tab, fu_tab,
          qoe_w, qoe_b, ch_w, ch_b, fu_w, fu_b,
          qoe_out, ch_out, fu_out,
          idxc, contc, idx_feat, gtmp, ctmp, wsc, bsc, sem):
    wid = lax.axis_index("s") * NC + lax.axis_index("c")
    base = wid * BW
    iot = lax.iota(jnp.int32, 16)

    groups = (
        (qoe_idx, qoe_cont, qoe_tab, qoe_w, qoe_b, qoe_out),
        (ch_idx, ch_cont, ch_tab, ch_w, ch_b, ch_out),
        (fu_idx, fu_cont, fu_tab, fu_w, fu_b, fu_out),
    )
    for idx2d, cont2d, tab, w, bia, out in groups:
        pltpu.sync_copy(idx2d.at[pl.ds(base, BW)], idxc)
        pltpu.sync_copy(cont2d.at[pl.ds(base * N_CONT, BW * N_CONT)], contc)
        pltpu.sync_copy(w, wsc)
        pltpu.sync_copy(bia, bsc)

        for i in range(N_DISC):
            off = 1 + i * V1
            cols = jnp.full((16,), i, jnp.int32)
            for k in range(NK):
                rows = k * 16 + iot
                v = plsc.load_gather(idxc, [rows, cols]) + off
                idx_feat[pl.ds(k * 16, 16)] = v
            pltpu.async_copy(tab.at[idx_feat], gtmp, sem).wait()
            pltpu.sync_copy(gtmp, out.at[pl.ds(base, BW), i])

        wlo = [wsc[j, pl.ds(0, 16)] for j in range(N_CONT)]
        whi = [wsc[j, pl.ds(16, 16)] for j in range(N_CONT)]
        blo = [bsc[j, pl.ds(0, 16)] for j in range(N_CONT)]
        bhi = [bsc[j, pl.ds(16, 16)] for j in range(N_CONT)]

        def cbody(q, carry):
            cvec = contc[pl.ds(q * 16, 16)]
            for rr in range(4):
                r = q * 4 + rr
                for j in range(N_CONT):
                    c = cvec[rr * N_CONT + j]
                    ctmp[r, j, pl.ds(0, 16)] = c * wlo[j] + blo[j]
                    ctmp[r, j, pl.ds(16, 16)] = c * whi[j] + bhi[j]
            return carry

        lax.fori_loop(0, BW * N_CONT // 16, cbody, None)
        pltpu.sync_copy(ctmp, out.at[pl.ds(base, BW), pl.ds(N_DISC, N_CONT)])


@jax.jit
def _impl(qoe_d, ch_d, fu_d, qoe_c, ch_c, fu_c,
          qoe_tab, ch_tab, fu_tab,
          qoe_w, qoe_b, ch_w, ch_b, fu_w, fu_b):
    mesh = plsc.VectorSubcoreMesh(core_axis_name="c", subcore_axis_name="s")
    out_t = [jax.ShapeDtypeStruct((B, N_FEAT, D), jnp.float32)] * 3
    scratch = [
        pltpu.VMEM((BW, N_DISC), jnp.int32),
        pltpu.VMEM((BW * N_CONT,), jnp.float32),
        pltpu.VMEM((BW,), jnp.int32),
        pltpu.VMEM((BW, D), jnp.float32),
        pltpu.VMEM((BW, N_CONT, D), jnp.float32),
        pltpu.VMEM((N_CONT, D), jnp.float32),
        pltpu.VMEM((N_CONT, D), jnp.float32),
        pltpu.SemaphoreType.DMA,
    ]
    run = pl.kernel(_body, mesh=mesh, out_type=out_t, scratch_types=scratch,
                    compiler_params=pltpu.CompilerParams(
                        use_tc_tiling_on_sc=False,
                        needs_layout_passes=False))
    outs = run(
        qoe_d.reshape(B, N_DISC), ch_d.reshape(B, N_DISC), fu_d.reshape(B, N_DISC),
        qoe_c.reshape(B * N_CONT), ch_c.reshape(B * N_CONT), fu_c.reshape(B * N_CONT),
        qoe_tab.reshape(N_DISC * V1, D), ch_tab.reshape(N_DISC * V1, D),
        fu_tab.reshape(N_DISC * V1, D),
        qoe_w, qoe_b, ch_w, ch_b, fu_w, fu_b,
    )
    return tuple(o.reshape(B, 1, N_FEAT, D) for o in outs)


def kernel(batch_feature_tensor_target_QOE_discrete,
           batch_feature_tensor_target_CHONGHE_discrete,
           batch_feature_tensor_target_FUFEI_discrete,
           batch_feature_tensor_target_QOE_continue,
           batch_feature_tensor_target_CHONGHE_continue,
           batch_feature_tensor_target_FUFEI_continue,
           qoe_tables, chonghe_tables, fufei_tables,
           qoe_cont_w, qoe_cont_b, chonghe_cont_w, chonghe_cont_b,
           fufei_cont_w, fufei_cont_b):
    return _impl(batch_feature_tensor_target_QOE_discrete,
                 batch_feature_tensor_target_CHONGHE_discrete,
                 batch_feature_tensor_target_FUFEI_discrete,
                 batch_feature_tensor_target_QOE_continue,
                 batch_feature_tensor_target_CHONGHE_continue,
                 batch_feature_tensor_target_FUFEI_continue,
                 qoe_tables, chonghe_tables, fufei_tables,
                 qoe_cont_w, qoe_cont_b, chonghe_cont_w, chonghe_cont_b,
                 fufei_cont_w, fufei_cont_b)
